# baseline (speedup 1.0000x reference)
import functools

import jax
import jax.numpy as jnp
from jax.experimental import pallas as pl
from jax.experimental.pallas import tpu as pltpu


def _se_gate_kernel(x_ref, w1t_hbm, b1_hbm, w2_hbm, b2_hbm, o_ref,
                    w1t_s, b1_s, w2_s, b2_s, sem, *, inv_hw):
    i = pl.program_id(0)

    @pl.when(i == 0)
    def _load_weights():
        c0 = pltpu.make_async_copy(w1t_hbm, w1t_s, sem.at[0])
        c1 = pltpu.make_async_copy(b1_hbm, b1_s, sem.at[1])
        c2 = pltpu.make_async_copy(w2_hbm, w2_s, sem.at[2])
        c3 = pltpu.make_async_copy(b2_hbm, b2_s, sem.at[3])
        c0.start(); c1.start(); c2.start(); c3.start()
        c0.wait(); c1.wait(); c2.wait(); c3.wait()

    pooled = jnp.sum(x_ref[...].astype(jnp.float32), axis=0) * inv_hw
    h = jax.lax.dot_general(pooled, w1t_s[...].astype(jnp.float32),
                            (((1,), (1,)), ((), ())),
                            preferred_element_type=jnp.float32) + b1_s[...]
    h = jnp.maximum(h, 0.0)
    g = jnp.dot(h, w2_s[...].astype(jnp.float32),
                preferred_element_type=jnp.float32) + b2_s[...]
    o_ref[...] = jnp.broadcast_to(g[None], o_ref.shape).astype(o_ref.dtype)


def _pick_tile(n, target):
    best = 1
    for d in range(1, n + 1):
        if n % d == 0 and d <= target:
            best = d
    return best


def kernel(x, w1, b1, w2, b2):
    B, C, H, W = x.shape
    HW = H * W
    Ch = w1.shape[1]
    itemsize = jnp.dtype(x.dtype).itemsize

    xt = jnp.transpose(x, (2, 3, 0, 1)).reshape(HW, B, C)
    w1t = jnp.transpose(w1)
    b1_2d = b1.reshape(1, Ch).astype(jnp.float32)
    b2_2d = b2.reshape(1, C).astype(jnp.float32)

    tb = _pick_tile(B, max(1, (56 << 20) // max(1, 4 * HW * C * itemsize)))
    nb = B // tb

    wspec = pl.BlockSpec(memory_space=pl.ANY)
    out = pl.pallas_call(
        functools.partial(_se_gate_kernel, inv_hw=1.0 / HW),
        out_shape=jax.ShapeDtypeStruct((HW, B, C), x.dtype),
        grid=(nb,),
        in_specs=[
            pl.BlockSpec((HW, tb, C), lambda i: (0, i, 0)),
            wspec, wspec, wspec, wspec,
        ],
        out_specs=pl.BlockSpec((HW, tb, C), lambda i: (0, i, 0)),
        scratch_shapes=[
            pltpu.VMEM((Ch, C), jnp.float32),
            pltpu.VMEM((1, Ch), jnp.float32),
            pltpu.VMEM((Ch, C), jnp.float32),
            pltpu.VMEM((1, C), jnp.float32),
            pltpu.SemaphoreType.DMA((4,)),
        ],
        compiler_params=pltpu.CompilerParams(
            dimension_semantics=("arbitrary",),
            vmem_limit_bytes=64 << 20),
        cost_estimate=pl.CostEstimate(
            flops=B * C * HW + 4 * B * C * Ch,
            transcendentals=0,
            bytes_accessed=2 * B * C * HW * itemsize),
    )(xt, w1t, b1_2d, w2, b2_2d)

    return jnp.transpose(out.reshape(H, W, B, C), (2, 3, 0, 1))


# speedup vs baseline: 1.0158x; 1.0158x over previous
import functools

import jax
import jax.numpy as jnp
from jax.experimental import pallas as pl
from jax.experimental.pallas import tpu as pltpu


def _se_gate_kernel(x_ref, w1t_hbm, b1_hbm, w2_hbm, b2_hbm, o_ref,
                    acc_ref, w1t_s, b1_s, w2_s, b2_s, sem, *, inv_hw):
    i = pl.program_id(0)
    nb = pl.num_programs(0) - 1
    cur = jax.lax.rem(i, 2)
    prev = jax.lax.rem(i + 1, 2)

    @pl.when(i == 0)
    def _load_weights():
        c0 = pltpu.make_async_copy(w1t_hbm, w1t_s, sem.at[0])
        c1 = pltpu.make_async_copy(b1_hbm, b1_s, sem.at[1])
        c2 = pltpu.make_async_copy(w2_hbm, w2_s, sem.at[2])
        c3 = pltpu.make_async_copy(b2_hbm, b2_s, sem.at[3])
        c0.start(); c1.start(); c2.start(); c3.start()
        c0.wait(); c1.wait(); c2.wait(); c3.wait()

    @pl.when(i >= 1)
    def _store():
        o_ref[...] = jnp.broadcast_to(
            acc_ref[pl.ds(prev, 1)], o_ref.shape).astype(o_ref.dtype)

    @pl.when(i < nb)
    def _pool_mlp():
        pooled = jnp.sum(x_ref[...].astype(jnp.float32), axis=0) * inv_hw
        h = jax.lax.dot_general(pooled, w1t_s[...].astype(jnp.float32),
                                (((1,), (1,)), ((), ())),
                                preferred_element_type=jnp.float32) + b1_s[...]
        h = jnp.maximum(h, 0.0)
        g = jnp.dot(h, w2_s[...].astype(jnp.float32),
                    preferred_element_type=jnp.float32) + b2_s[...]
        acc_ref[pl.ds(cur, 1)] = g[None]


def _pick_tile(n, target):
    best = 1
    for d in range(1, n + 1):
        if n % d == 0 and d <= target:
            best = d
    return best


def kernel(x, w1, b1, w2, b2):
    B, C, H, W = x.shape
    HW = H * W
    Ch = w1.shape[1]
    itemsize = jnp.dtype(x.dtype).itemsize

    xt = jnp.transpose(x, (2, 3, 0, 1)).reshape(HW, B, C)
    w1t = jnp.transpose(w1)
    b1_2d = b1.reshape(1, Ch).astype(jnp.float32)
    b2_2d = b2.reshape(1, C).astype(jnp.float32)

    tb = _pick_tile(B, max(1, (56 << 20) // max(1, 4 * HW * C * itemsize)))
    nb = B // tb

    wspec = pl.BlockSpec(memory_space=pl.ANY)
    out = pl.pallas_call(
        functools.partial(_se_gate_kernel, inv_hw=1.0 / HW),
        out_shape=jax.ShapeDtypeStruct((HW, B, C), x.dtype),
        grid=(nb + 1,),
        in_specs=[
            pl.BlockSpec((HW, tb, C), lambda i: (0, jnp.minimum(i, nb - 1), 0)),
            wspec, wspec, wspec, wspec,
        ],
        out_specs=pl.BlockSpec(
            (HW, tb, C), lambda i: (0, jnp.maximum(i - 1, 0), 0)),
        scratch_shapes=[
            pltpu.VMEM((2, tb, C), jnp.float32),
            pltpu.VMEM((Ch, C), jnp.float32),
            pltpu.VMEM((1, Ch), jnp.float32),
            pltpu.VMEM((Ch, C), jnp.float32),
            pltpu.VMEM((1, C), jnp.float32),
            pltpu.SemaphoreType.DMA((4,)),
        ],
        compiler_params=pltpu.CompilerParams(
            dimension_semantics=("arbitrary",),
            vmem_limit_bytes=64 << 20),
        cost_estimate=pl.CostEstimate(
            flops=B * C * HW + 4 * B * C * Ch,
            transcendentals=0,
            bytes_accessed=2 * B * C * HW * itemsize),
    )(xt, w1t, b1_2d, w2, b2_2d)

    return jnp.transpose(out.reshape(H, W, B, C), (2, 3, 0, 1))


# speedup vs baseline: 1.0520x; 1.0356x over previous
import functools

import jax
import jax.numpy as jnp
from jax.experimental import pallas as pl
from jax.experimental.pallas import tpu as pltpu


def _se_gate_kernel(x_ref, w1t_ref, b1_ref, w2_ref, b2_ref, o_ref, acc_ref,
                    *, inv_hw):
    i = pl.program_id(0)
    nb = pl.num_programs(0) - 1
    cur = jax.lax.rem(i, 2)
    prev = jax.lax.rem(i + 1, 2)

    @pl.when(i >= 1)
    def _store():
        o_ref[...] = jnp.broadcast_to(
            acc_ref[pl.ds(prev, 1)], o_ref.shape).astype(o_ref.dtype)

    @pl.when(i < nb)
    def _pool_mlp():
        pooled = jnp.sum(x_ref[...].astype(jnp.float32), axis=0) * inv_hw
        h = jax.lax.dot_general(pooled, w1t_ref[...].astype(jnp.float32),
                                (((1,), (1,)), ((), ())),
                                preferred_element_type=jnp.float32) + b1_ref[...]
        h = jnp.maximum(h, 0.0)
        g = jnp.dot(h, w2_ref[...].astype(jnp.float32),
                    preferred_element_type=jnp.float32) + b2_ref[...]
        acc_ref[pl.ds(cur, 1)] = g[None]


def _pick_tile(n, target):
    best = 1
    for d in range(1, n + 1):
        if n % d == 0 and d <= target:
            best = d
    return best


def kernel(x, w1, b1, w2, b2):
    B, C, H, W = x.shape
    HW = H * W
    Ch = w1.shape[1]
    itemsize = jnp.dtype(x.dtype).itemsize

    xt = jnp.transpose(x, (2, 3, 0, 1)).reshape(HW, B, C)
    w1t = jnp.transpose(w1)
    b1_2d = b1.reshape(1, Ch).astype(jnp.float32)
    b2_2d = b2.reshape(1, C).astype(jnp.float32)

    tb = _pick_tile(B, max(1, (56 << 20) // max(1, 4 * HW * C * itemsize)))
    nb = B // tb

    out = pl.pallas_call(
        functools.partial(_se_gate_kernel, inv_hw=1.0 / HW),
        out_shape=jax.ShapeDtypeStruct((HW, B, C), x.dtype),
        grid=(nb + 1,),
        in_specs=[
            pl.BlockSpec((HW, tb, C), lambda i: (0, jnp.minimum(i, nb - 1), 0)),
            pl.BlockSpec((Ch, C), lambda i: (0, 0)),
            pl.BlockSpec((1, Ch), lambda i: (0, 0)),
            pl.BlockSpec((Ch, C), lambda i: (0, 0)),
            pl.BlockSpec((1, C), lambda i: (0, 0)),
        ],
        out_specs=pl.BlockSpec(
            (HW, tb, C), lambda i: (0, jnp.maximum(i - 1, 0), 0)),
        scratch_shapes=[pltpu.VMEM((2, tb, C), jnp.float32)],
        compiler_params=pltpu.CompilerParams(
            dimension_semantics=("arbitrary",),
            vmem_limit_bytes=64 << 20),
        cost_estimate=pl.CostEstimate(
            flops=B * C * HW + 4 * B * C * Ch,
            transcendentals=0,
            bytes_accessed=2 * B * C * HW * itemsize),
    )(xt, w1t, b1_2d, w2, b2_2d)

    return jnp.transpose(out.reshape(H, W, B, C), (2, 3, 0, 1))
